# baseline (speedup 1.0000x reference)
import jax
import jax.numpy as jnp
from jax.experimental import pallas as pl
from jax.experimental.pallas import tpu as pltpu

_N_ITER = 5
_G = 8


def _routing_body(u_ref, bt_ref, v_ref, s1_ref):
    g = _G
    ocaps, icaps = bt_ref.shape
    od = u_ref.shape[1]
    dim = od // ocaps
    u = u_ref[...]
    u4 = u.reshape(g, ocaps, dim, icaps)
    ub = u.astype(jnp.bfloat16)
    ko = jax.lax.broadcasted_iota(jnp.int32, (ocaps, od), 1) // dim
    oo = jax.lax.broadcasted_iota(jnp.int32, (ocaps, od), 0)
    omask = (ko == oo).astype(jnp.float32)

    b = jnp.broadcast_to(bt_ref[...][None], (g, ocaps, icaps))
    s14 = None
    vcol4 = None
    for r in range(_N_ITER):
        if r > 0:
            vrow = jnp.swapaxes(vcol4.reshape(g, od, 1), 1, 2)
            w = (omask[None] * vrow).astype(jnp.bfloat16)
            z = jax.lax.dot_general(
                w, ub, (((2,), (1,)), ((0,), (0,))),
                preferred_element_type=jnp.float32)
            b = b + z
        e = jnp.exp(b)
        rs = jax.lax.reciprocal(jnp.sum(e, axis=1, keepdims=True))
        c = e * rs
        if r == _N_ITER - 1:
            s14 = u4 * c[:, :, None, :]
            s24 = jnp.sum(s14, axis=3, keepdims=True)
        else:
            s24 = jnp.sum(u4 * c[:, :, None, :], axis=3, keepdims=True)
        n2 = jnp.sum(s24 * s24, axis=2, keepdims=True)
        scale = jnp.sqrt(n2) * jax.lax.reciprocal(1.0 + n2)
        vcol4 = s24 * scale
    v_ref[...] = vcol4.reshape(g, od, 1)
    s1_ref[...] = s14.reshape(g, od, icaps)


def kernel(u_predict, b):
    bsz, icaps, ocaps, dim = u_predict.shape
    od = ocaps * dim
    u_t = u_predict.transpose(0, 2, 3, 1).reshape(bsz, od, icaps)
    b_t = b.T
    v_t, s1_t = pl.pallas_call(
        _routing_body,
        grid=(bsz // _G,),
        in_specs=[
            pl.BlockSpec((_G, od, icaps), lambda i: (i, 0, 0)),
            pl.BlockSpec((ocaps, icaps), lambda i: (0, 0)),
        ],
        out_specs=[
            pl.BlockSpec((_G, od, 1), lambda i: (i, 0, 0)),
            pl.BlockSpec((_G, od, icaps), lambda i: (i, 0, 0)),
        ],
        out_shape=[
            jax.ShapeDtypeStruct((bsz, od, 1), jnp.float32),
            jax.ShapeDtypeStruct((bsz, od, icaps), jnp.float32),
        ],
        compiler_params=pltpu.CompilerParams(
            dimension_semantics=("parallel",),
        ),
    )(u_t, b_t)
    v = v_t.reshape(bsz, ocaps, dim)
    s1 = s1_t.reshape(bsz, ocaps, dim, icaps).transpose(0, 3, 1, 2)
    return v, s1
